# Initial kernel scaffold; baseline (speedup 1.0000x reference)
import jax
import jax.numpy as jnp
from jax.experimental import pallas as pl


def kernel(expert_indices, expert_embeddings_weight):
    raise NotImplementedError("write your pallas kernel here")


# speedup vs baseline: 1.2466x; 1.2466x over previous
import functools

import jax
import jax.numpy as jnp
from jax import lax
from jax.experimental import pallas as pl
from jax.experimental.pallas import tpu as pltpu
from jax.experimental.pallas import tpu_sc as plsc

NUM_EXPERTS = 64
EMBED_DIM = 2048
N_TOKENS = 16384
TOP_K = 8

_NC, _NS = 2, 16
_NW = _NC * _NS
_B = N_TOKENS * TOP_K
_B_PER_W = _B // _NW
_CHUNK = 32
_NCHUNK = _B_PER_W // _CHUNK


def _sc_gather(idx_flat, table):
    mesh = plsc.VectorSubcoreMesh(core_axis_name="c", subcore_axis_name="s")

    @functools.partial(
        pl.kernel,
        out_type=jax.ShapeDtypeStruct((_B, EMBED_DIM), jnp.float32),
        mesh=mesh,
        scratch_types=[
            pltpu.VMEM((_B_PER_W,), jnp.int32),
            pltpu.VMEM((_CHUNK, EMBED_DIM), jnp.float32),
        ],
    )
    def k(table_hbm, idx_hbm, out_hbm, idx_v, rows_v):
        wid = lax.axis_index("s") * _NC + lax.axis_index("c")
        base = wid * _B_PER_W
        pltpu.sync_copy(idx_hbm.at[pl.ds(base, _B_PER_W)], idx_v)

        @pl.loop(0, _NCHUNK)
        def _(i):
            start = i * _CHUNK
            pltpu.sync_copy(table_hbm.at[idx_v.at[pl.ds(start, _CHUNK)]],
                            rows_v)
            pltpu.sync_copy(rows_v, out_hbm.at[pl.ds(base + start, _CHUNK)])

    return k(table, idx_flat)


def kernel(expert_indices, expert_embeddings_weight):
    idx = expert_indices.reshape(-1).astype(jnp.int32)
    out = _sc_gather(idx, expert_embeddings_weight)
    return out.reshape(N_TOKENS, TOP_K, EMBED_DIM)
